# baseline (speedup 1.0000x reference)
import functools

import jax
import jax.numpy as jnp
from jax import lax
from jax.experimental import pallas as pl
from jax.experimental.pallas import tpu as pltpu
from jax.experimental.pallas import tpu_sc as plsc


def _make_sc_lookup(B, V, D):
    info = plsc.get_sparse_core_info()
    NW = info.num_cores * info.num_subcores
    b_per_w = B // NW
    assert B % NW == 0 and b_per_w % 8 == 0

    mesh = plsc.VectorSubcoreMesh(core_axis_name="c", subcore_axis_name="s")

    @functools.partial(
        pl.kernel,
        out_type=jax.ShapeDtypeStruct((B, D), jnp.float32),
        mesh=mesh,
        compiler_params=pltpu.CompilerParams(use_tc_tiling_on_sc=False),
        scratch_types=[
            pltpu.VMEM((b_per_w,), jnp.int32),
            pltpu.VMEM((b_per_w,), jnp.int32),
            pltpu.VMEM((b_per_w, D), jnp.float32),
            pltpu.VMEM((b_per_w, D), jnp.float32),
            pltpu.SemaphoreType.DMA,
        ],
    )
    def run(idx0_hbm, idx1_hbm, w0_hbm, w1_hbm, out_hbm,
            idx0_v, idx1_v, rows0_v, rows1_v, sem):
        nc = info.num_cores
        wid = lax.axis_index("s") * nc + lax.axis_index("c")
        base = wid * b_per_w
        pltpu.sync_copy(idx0_hbm.at[pl.ds(base, b_per_w)], idx0_v)
        pltpu.sync_copy(idx1_hbm.at[pl.ds(base, b_per_w)], idx1_v)
        cp0 = pltpu.async_copy(w0_hbm.at[idx0_v], rows0_v, sem)
        cp1 = pltpu.async_copy(w1_hbm.at[idx1_v], rows1_v, sem)
        cp0.wait()
        cp1.wait()

        def add_rows(i, carry):
            for j in range(D // 16):
                sl = pl.ds(j * 16, 16)
                rows0_v[i, sl] = rows0_v[i, sl] + rows1_v[i, sl]
            return carry

        lax.fori_loop(0, b_per_w, add_rows, 0, unroll=8)

        pltpu.sync_copy(rows0_v, out_hbm.at[pl.ds(base, b_per_w)])

    return run


def kernel(x, W0, W1):
    B = x.shape[0]
    V, D = W0.shape
    idx0 = x[:, 0].astype(jnp.int32)
    idx1 = x[:, 1].astype(jnp.int32)
    return _make_sc_lookup(B, V, D)(idx0, idx1, W0, W1)
